# baseline (speedup 1.0000x reference)
import jax
import jax.numpy as jnp
from jax.experimental import pallas as pl

BATCH = 16
MAX_LAB = 14
NROW = 4
IMG_W = 640.0
IMG_H = 640.0
N_DET = 20000
N_GT = BATCH * MAX_LAB
CHUNK = 1024
TAIL = N_DET - 19 * CHUNK
BIG = 3.0e9


def _tc_body(people_ref, imgcol_ref, pred_ref, cls_ref, out_ref, aid_ref):
    imgcol = imgcol_ref[...]
    giota = jax.lax.broadcasted_iota(jnp.int32, (N_GT, 1), 0)
    lab = giota % MAX_LAB
    slot = giota // MAX_LAB
    ohsmall = (jax.lax.broadcasted_iota(jnp.int32, (BATCH, BATCH), 1)
               == (imgcol % BATCH)).astype(jnp.float32)
    repsel = (jax.lax.broadcasted_iota(jnp.int32, (N_GT, BATCH), 1)
              == slot).astype(jnp.float32)
    ohT = jnp.dot(repsel, ohsmall, preferred_element_type=jnp.float32)
    imgrep = jnp.dot(repsel, imgcol.astype(jnp.float32),
                     preferred_element_type=jnp.float32)
    sel56 = jnp.dot(ohT, people_ref[...], preferred_element_type=jnp.float32)

    lane56 = jax.lax.broadcasted_iota(jnp.int32, (N_GT, 4 * MAX_LAB), 1)
    labm = (lane56 // 4 == lab).astype(jnp.float32)
    c_l = lane56 % 4

    def pickc(c):
        mc = labm * (c_l == c).astype(jnp.float32)
        return jnp.sum(sel56 * mc, axis=1, keepdims=True)

    cxs = pickc(0)
    cys = pickc(1)
    ws = pickc(2)
    hs = pickc(3)

    offx = IMG_W * (slot % NROW).astype(jnp.float32)
    offy = IMG_H * (slot // NROW).astype(jnp.float32)
    nz = (cxs + cys + ws + hs) != 0.0
    cxp = cxs * IMG_W + jnp.where(nz, offx, 0.0)
    cyp = cys * IMG_H + jnp.where(nz, offy, 0.0)
    bw = IMG_W * ws
    bh = IMG_H * hs
    x1 = cxp - bw * 0.5
    y1 = cyp - bh * 0.5
    x2 = x1 + bw
    y2 = y1 + bh
    needed = (x1 + y1 + x2 + y2) != 0.0
    c_x1 = jnp.where(needed, x1, 0.0)
    c_y1 = jnp.where(needed, y1, 0.0)
    c_x2 = jnp.where(needed, x2, 0.0)
    c_y2 = jnp.where(needed, y2, 0.0)
    aidc = jnp.where(needed & (imgrep >= float(BATCH)), 1, 0).astype(jnp.int32)
    c_ag = (c_x2 - c_x1 + 1.0) * (c_y2 - c_y1 + 1.0)

    m = cls_ref[...] == 0
    anyb = jnp.max(m.astype(jnp.float32)) > 0.0

    zrow = jnp.zeros((1, CHUNK), jnp.float32)
    gx1b = c_x1 + zrow
    gy1b = c_y1 + zrow
    gx2b = (c_x2 + 1.0) + zrow
    gy2b = (c_y2 + 1.0) + zrow
    gagb = c_ag + zrow

    def chunk_body(start, width, acc):
        sl = pl.ds(start, width)
        px1 = jnp.where(cls_ref[:, sl] == 0, pred_ref[0:1, sl], BIG)
        py1 = pred_ref[1:2, sl]
        px2p = pred_ref[2:3, sl] + 1.0
        py2p = pred_ref[3:4, sl] + 1.0
        areab = (px2p - pred_ref[0:1, sl]) * (py2p - py1)
        iw = jnp.maximum(jnp.minimum(gx2b[:, :width], px2p)
                         - jnp.maximum(gx1b[:, :width], px1), 0.0)
        ih = jnp.maximum(jnp.minimum(gy2b[:, :width], py2p)
                         - jnp.maximum(gy1b[:, :width], py1), 0.0)
        inters = iw * ih
        uni = gagb[:, :width] + areab - inters
        return jnp.maximum(acc, jnp.max(inters / uni, axis=1, keepdims=True))

    ov = jnp.zeros((N_GT, 1), jnp.float32)
    for c in range(19):
        ov = chunk_body(c * CHUNK, CHUNK, ov)
    ov = chunk_body(19 * CHUNK, TAIL, ov)

    ovr = jnp.transpose(ov)
    aidf = jnp.transpose(aidc).astype(jnp.float32)
    iou_pred = jnp.concatenate([ovr, 1.0 - ovr], axis=0)
    basep = jnp.concatenate([aidf, jnp.abs(aidf - 1.0)], axis=0)
    out_ref[...] = jnp.where(anyb, iou_pred, basep) * 10.0
    aid_ref[...] = jnp.transpose(aidc)


def kernel(people_boxes, pred_boxes, pred_scores, pred_classes, image_index):
    del pred_scores
    people56 = people_boxes.reshape(BATCH, 4 * MAX_LAB)
    imgcol = image_index.reshape(BATCH, 1)
    coords = pred_boxes.T
    clsrow = pred_classes.reshape(1, N_DET)

    predict, aid = pl.pallas_call(
        _tc_body,
        out_shape=[
            jax.ShapeDtypeStruct((2, N_GT), jnp.float32),
            jax.ShapeDtypeStruct((1, N_GT), jnp.int32),
        ],
    )(people56, imgcol, coords, clsrow)
    return (predict.T, aid.reshape(N_GT))


# speedup vs baseline: 1.0056x; 1.0056x over previous
import jax
import jax.numpy as jnp
from jax.experimental import pallas as pl

BATCH = 16
MAX_LAB = 14
NROW = 4
IMG_W = 640.0
IMG_H = 640.0
N_DET = 20000
N_GT = BATCH * MAX_LAB
CHUNK = 512
TAIL = N_DET - 39 * CHUNK
BIG = 3.0e9


def _tc_body(people_ref, imgcol_ref, pred_ref, cls_ref, out_ref, aid_ref):
    imgcol = imgcol_ref[...]
    giota = jax.lax.broadcasted_iota(jnp.int32, (N_GT, 1), 0)
    lab = giota % MAX_LAB
    slot = giota // MAX_LAB
    ohsmall = (jax.lax.broadcasted_iota(jnp.int32, (BATCH, BATCH), 1)
               == (imgcol % BATCH)).astype(jnp.float32)
    repsel = (jax.lax.broadcasted_iota(jnp.int32, (N_GT, BATCH), 1)
              == slot).astype(jnp.float32)
    ohT = jnp.dot(repsel, ohsmall, preferred_element_type=jnp.float32)
    imgrep = jnp.dot(repsel, imgcol.astype(jnp.float32),
                     preferred_element_type=jnp.float32)
    sel56 = jnp.dot(ohT, people_ref[...], preferred_element_type=jnp.float32)

    lane56 = jax.lax.broadcasted_iota(jnp.int32, (N_GT, 4 * MAX_LAB), 1)
    labm = (lane56 // 4 == lab).astype(jnp.float32)
    c_l = lane56 % 4

    def pickc(c):
        mc = labm * (c_l == c).astype(jnp.float32)
        return jnp.sum(sel56 * mc, axis=1, keepdims=True)

    cxs = pickc(0)
    cys = pickc(1)
    ws = pickc(2)
    hs = pickc(3)

    offx = IMG_W * (slot % NROW).astype(jnp.float32)
    offy = IMG_H * (slot // NROW).astype(jnp.float32)
    nz = (cxs + cys + ws + hs) != 0.0
    cxp = cxs * IMG_W + jnp.where(nz, offx, 0.0)
    cyp = cys * IMG_H + jnp.where(nz, offy, 0.0)
    bw = IMG_W * ws
    bh = IMG_H * hs
    x1 = cxp - bw * 0.5
    y1 = cyp - bh * 0.5
    x2 = x1 + bw
    y2 = y1 + bh
    needed = (x1 + y1 + x2 + y2) != 0.0
    c_x1 = jnp.where(needed, x1, 0.0)
    c_y1 = jnp.where(needed, y1, 0.0)
    c_x2 = jnp.where(needed, x2, 0.0)
    c_y2 = jnp.where(needed, y2, 0.0)
    aidc = jnp.where(needed & (imgrep >= float(BATCH)), 1, 0).astype(jnp.int32)
    c_ag = (c_x2 - c_x1 + 1.0) * (c_y2 - c_y1 + 1.0)

    m = cls_ref[...] == 0
    anyb = jnp.max(m.astype(jnp.float32)) > 0.0

    zrow = jnp.zeros((1, CHUNK), jnp.float32)
    gx1b = c_x1 + zrow
    gy1b = c_y1 + zrow
    gx2b = (c_x2 + 1.0) + zrow
    gy2b = (c_y2 + 1.0) + zrow
    gagb = c_ag + zrow

    def chunk_body(start, width, acc):
        sl = pl.ds(start, width)
        px1 = jnp.where(cls_ref[:, sl] == 0, pred_ref[0:1, sl], BIG)
        py1 = pred_ref[1:2, sl]
        px2p = pred_ref[2:3, sl] + 1.0
        py2p = pred_ref[3:4, sl] + 1.0
        areab = (px2p - pred_ref[0:1, sl]) * (py2p - py1)
        iw = jnp.maximum(jnp.minimum(gx2b[:, :width], px2p)
                         - jnp.maximum(gx1b[:, :width], px1), 0.0)
        ih = jnp.maximum(jnp.minimum(gy2b[:, :width], py2p)
                         - jnp.maximum(gy1b[:, :width], py1), 0.0)
        inters = iw * ih
        uni = gagb[:, :width] + areab - inters
        return jnp.maximum(acc, jnp.max(inters / uni, axis=1, keepdims=True))

    ov = jnp.zeros((N_GT, 1), jnp.float32)
    for c in range(39):
        ov = chunk_body(c * CHUNK, CHUNK, ov)
    ov = chunk_body(39 * CHUNK, TAIL, ov)

    ovr = jnp.transpose(ov)
    aidf = jnp.transpose(aidc).astype(jnp.float32)
    iou_pred = jnp.concatenate([ovr, 1.0 - ovr], axis=0)
    basep = jnp.concatenate([aidf, jnp.abs(aidf - 1.0)], axis=0)
    out_ref[...] = jnp.where(anyb, iou_pred, basep) * 10.0
    aid_ref[...] = jnp.transpose(aidc)


def kernel(people_boxes, pred_boxes, pred_scores, pred_classes, image_index):
    del pred_scores
    people56 = people_boxes.reshape(BATCH, 4 * MAX_LAB)
    imgcol = image_index.reshape(BATCH, 1)
    coords = pred_boxes.T
    clsrow = pred_classes.reshape(1, N_DET)

    predict, aid = pl.pallas_call(
        _tc_body,
        out_shape=[
            jax.ShapeDtypeStruct((2, N_GT), jnp.float32),
            jax.ShapeDtypeStruct((1, N_GT), jnp.int32),
        ],
    )(people56, imgcol, coords, clsrow)
    return (predict.T, aid.reshape(N_GT))
